# baseline (speedup 1.0000x reference)
import functools

import numpy as np
import jax
import jax.numpy as jnp
from jax import lax
from jax.experimental import pallas as pl
from jax.experimental.pallas import tpu as pltpu
from jax.experimental.pallas import tpu_sc as plsc

_B = 32
_IN = 4096
_OUT = 4096
_K = 128
_EPS = 1e-6
_P1 = 32749
_P2 = 32719


def _threefry_raw(keypair, x0, x1):
    x0 = x0.copy()
    x1 = x1.copy()
    ks0 = np.uint32(keypair[0])
    ks1 = np.uint32(keypair[1])
    ks2 = np.uint32(ks0 ^ ks1 ^ np.uint32(0x1BD11BDA))
    rot = [np.uint32(r) for r in (13, 15, 26, 6, 17, 29, 16, 24)]

    def rotl(x, d):
        return (x << d) | (x >> np.uint32(32 - int(d)))

    x0 += ks0
    x1 += ks1
    ks = [ks1, ks2, ks2, ks0, ks0, ks1, ks1, ks2, ks2, ks0]
    for g in range(5):
        for r in (rot[0:4] if g % 2 == 0 else rot[4:8]):
            x0 += x1
            x1 = rotl(x1, r)
            x1 ^= x0
        x0 += ks[2 * g]
        x1 += ks[2 * g + 1] + np.uint32(g + 1)
    return x0, x1


def _uniform01(keypair, n):
    b1, b2 = _threefry_raw(keypair, np.zeros(n, np.uint32),
                           np.arange(n, dtype=np.uint32))
    bits = b1 ^ b2
    return ((bits >> np.uint32(9)) | np.uint32(0x3F800000)).view(np.float32) \
        - np.float32(1.0)


def _gen_consts():
    err = np.seterr(over="ignore")
    key1 = np.array([0, 1], np.uint32)
    s1, s2 = _threefry_raw(key1, np.zeros(2, np.uint32),
                           np.arange(2, dtype=np.uint32))
    kr = np.array([s1[0], s2[0]], np.uint32)
    kg = np.array([s1[1], s2[1]], np.uint32)
    n = _B * _K * 4 * 2
    rr = (_uniform01(kr, n) * np.float32(1.0 - _EPS)).reshape(_B, _K, 4, 2)
    gs = (_uniform01(kg, n) * np.float32(1.0 - _EPS)).reshape(_B, _K, 4, 2)
    np.seterr(**err)
    gsf = gs * np.float32(4096.0)
    smp = np.floor(gsf).astype(np.int32)
    return (
        np.ascontiguousarray(rr[..., 0].transpose(0, 2, 1)),
        np.ascontiguousarray(rr[..., 1].transpose(0, 2, 1)),
        np.ascontiguousarray(smp[..., 0].transpose(0, 2, 1)),
        np.ascontiguousarray(smp[..., 1].transpose(0, 2, 1)),
    )


_RR_R, _RR_C, _SMP_R, _SMP_C = _gen_consts()


def _hyper_body(inp_ref, w_ref, b_ref, mr_ref, mc_ref, inv_ref, val_ref):
    res = jnp.dot(inp_ref[...], w_ref[...], preferred_element_type=jnp.float32)
    res = res + b_ref[...]
    x_mr = res[:, 0:128]
    x_mc = res[:, 128:256]
    x_sg = res[:, 256:384]
    x_v = res[:, 384:512]
    mr_ref[...] = 4095.0 / (1.0 + jnp.exp(-x_mr))
    mc_ref[...] = 4095.0 / (1.0 + jnp.exp(-x_mc))
    xs = x_sg + 2.0
    sp = jnp.maximum(xs, 0.0) + jnp.log(1.0 + jnp.exp(-jnp.abs(xs)))
    sig = (sp + _EPS) * 4096.0
    inv_ref[...] = 1.0 / (_EPS + sig)
    val_ref[...] = x_v


def _points_body(mr_ref, mc_ref, inv_ref, val_ref, sr_ref, sc_ref, rr_ref, rc_ref,
                 rows_ref, cols_ref, vals_ref, props_scr):
    b = pl.program_id(0)
    mr = mr_ref[0]
    mc = mc_ref[0]
    inv = inv_ref[0]
    w0 = val_ref[0]

    fr = jnp.floor(mr)
    cr = jnp.ceil(mr)
    fc = jnp.floor(mc)
    cc = jnp.ceil(mc)
    mnsr = jnp.round(mr)
    mnsc = jnp.round(mc)
    lo_r = jnp.maximum(mnsr - 32.0, 0.0)
    lo_r = jnp.where(mnsr + 32.0 > 4096.0, 4096.0 - 64.0, lo_r)
    lo_c = jnp.maximum(mnsc - 32.0, 0.0)
    lo_c = jnp.where(mnsc + 32.0 > 4096.0, 4096.0 - 64.0, lo_c)
    reg_r = rr_ref[0] * 64.0 + lo_r
    reg_c = rc_ref[0] * 64.0 + lo_c

    nbr_r = jnp.concatenate([fr, fr, cr, cr], axis=0).astype(jnp.int32)
    nbr_c = jnp.concatenate([fc, cc, fc, cc], axis=0).astype(jnp.int32)
    R = jnp.concatenate([nbr_r, sr_ref[0], reg_r.astype(jnp.int32)], axis=0)
    C = jnp.concatenate([nbr_c, sc_ref[0], reg_c.astype(jnp.int32)], axis=0)

    rp = R + 1
    cp = C + 1
    k1 = (rp * rp) * ((cp * cp) * cp)
    a1 = (rp * rp) % _P1
    t1 = (((cp * cp) % _P1) * cp) % _P1
    k2 = (a1 * t1) % _P1
    a2 = (rp * rp) % _P2
    t2 = (((cp * cp) % _P2) * cp) % _P2
    k3 = (a2 * t2) % _P2
    k23 = k2 * 32768 + k3

    tri = lax.broadcasted_iota(jnp.int32, (_K, _K), 1) > \
        lax.broadcasted_iota(jnp.int32, (_K, _K), 0)
    k1l = [k1[s][None, :] for s in range(12)]
    k23l = [k23[s][None, :] for s in range(12)]
    k1s = [k1[s][:, None] for s in range(12)]
    k23s = [k23[s][:, None] for s in range(12)]
    dup_rows = []
    for s1 in range(12):
        acc = None
        for s2 in range(s1 + 1):
            eq = (k1l[s1] == k1s[s2]) & (k23l[s1] == k23s[s2])
            if s2 == s1:
                eq = eq & tri
            acc = eq if acc is None else (acc | eq)
        hit = jnp.where(acc, jnp.int32(1), jnp.int32(0))
        dup_rows.append(jnp.max(hit, axis=0))

    Rf = R.astype(jnp.float32)
    Cf = C.astype(jnp.float32)
    S = jnp.zeros((1, 128), jnp.float32)
    for s in range(12):
        dr = Rf[s][:, None] - mr
        dc = Cf[s][:, None] - mc
        p = jnp.exp(-0.5 * (dr * dr + dc * dc) * inv)
        p = jnp.where(dup_rows[s][:, None] > 0, 0.0, p)
        props_scr[s] = p
        S = S + jnp.sum(p, axis=0, keepdims=True)
    w = w0 / (S + _EPS)
    vlist = [jnp.sum(props_scr[s] * w, axis=1)[None, :] for s in range(12)]

    soff = (b // 2) * 4096
    zpad_i = jnp.zeros((4, _K), jnp.int32)
    zpad_f = jnp.zeros((4, _K), jnp.float32)
    rows_ref[...] = jnp.concatenate([R + soff, zpad_i + soff], axis=0)
    cols_ref[...] = jnp.concatenate([C + soff, zpad_i + soff], axis=0)
    vals_ref[...] = jnp.concatenate([jnp.concatenate(vlist, axis=0), zpad_f], axis=0)


def _sc_body(inp_hbm, rows_hbm, cols_hbm, vals_hbm, out_hbm,
             cols_v, rows_v, vals_v, gath_v, y_v, inp_sh, y_sh):
    c = lax.axis_index("c")
    s = lax.axis_index("s")
    b = s * jnp.int32(2) + c
    soff = s * jnp.int32(4096)
    bsl = pl.ds(b * jnp.int32(16), 16)
    pltpu.sync_copy(inp_hbm.at[b], inp_sh.at[pl.ds(soff, 4096)])
    pltpu.sync_copy(cols_hbm.at[bsl], cols_v)
    pltpu.sync_copy(rows_hbm.at[bsl], rows_v)
    pltpu.sync_copy(vals_hbm.at[bsl], vals_v)
    for j in range(12):
        pltpu.sync_copy(inp_sh.at[cols_v.at[jnp.int32(j)]],
                        gath_v.at[jnp.int32(j)])
    for j in range(12):
        for t in range(8):
            ix = (jnp.int32(j), pl.ds(t * 16, 16))
            gath_v[ix] = gath_v[ix] * vals_v[ix]

    def zero(i, carry):
        y_v[pl.ds(i * jnp.int32(16), 16)] = jnp.zeros((16,), jnp.float32)
        return carry

    lax.fori_loop(jnp.int32(0), jnp.int32(256), zero, jnp.int32(0))
    pltpu.sync_copy(y_v, y_sh.at[pl.ds(soff, 4096)])
    for j in range(12):
        pltpu.sync_copy(gath_v.at[jnp.int32(j)],
                        y_sh.at[rows_v.at[jnp.int32(j)]], add=True)
    pltpu.sync_copy(y_sh.at[pl.ds(soff, 4096)], out_hbm.at[b])


def kernel(input, W_hyper, b_hyper):
    inp = input.astype(jnp.float32)
    Wp = W_hyper.astype(jnp.float32).reshape(_IN, _K, 4).transpose(0, 2, 1).reshape(_IN, 4 * _K)
    bp = b_hyper.astype(jnp.float32).reshape(_K, 4).T.reshape(1, 4 * _K)

    mr, mc, inv, val = pl.pallas_call(
        _hyper_body,
        out_shape=[jax.ShapeDtypeStruct((_B, _K), jnp.float32)] * 4,
    )(inp, Wp, bp)

    sr = jnp.asarray(_SMP_R)
    sc = jnp.asarray(_SMP_C)
    rrr = jnp.asarray(_RR_R)
    rrc = jnp.asarray(_RR_C)

    _z = np.int32(0)
    vec_spec = pl.BlockSpec((1, 1, _K), lambda b: (b, _z, _z))
    slot_spec = pl.BlockSpec((1, 4, _K), lambda b: (b, _z, _z))
    out_spec = pl.BlockSpec((16, _K), lambda b: (b, _z))
    rows, cols, vals = pl.pallas_call(
        _points_body,
        grid=(_B,),
        in_specs=[vec_spec, vec_spec, vec_spec, vec_spec,
                  slot_spec, slot_spec, slot_spec, slot_spec],
        out_specs=[out_spec, out_spec, out_spec],
        out_shape=[
            jax.ShapeDtypeStruct((_B * 16, _K), jnp.int32),
            jax.ShapeDtypeStruct((_B * 16, _K), jnp.int32),
            jax.ShapeDtypeStruct((_B * 16, _K), jnp.float32),
        ],
        scratch_shapes=[pltpu.VMEM((12, _K, _K), jnp.float32)],
    )(mr.reshape(_B, 1, _K), mc.reshape(_B, 1, _K),
      inv.reshape(_B, 1, _K), val.reshape(_B, 1, _K), sr, sc, rrr, rrc)

    sc_call = functools.partial(
        pl.kernel,
        mesh=plsc.VectorSubcoreMesh(core_axis_name="c", subcore_axis_name="s"),
        out_type=jax.ShapeDtypeStruct((_B, _OUT), jnp.float32),
        scratch_types=[
            pltpu.VMEM((16, _K), jnp.int32),
            pltpu.VMEM((16, _K), jnp.int32),
            pltpu.VMEM((16, _K), jnp.float32),
            pltpu.VMEM((12, _K), jnp.float32),
            pltpu.VMEM((_OUT,), jnp.float32),
            pltpu.VMEM_SHARED((16 * _IN,), jnp.float32),
            pltpu.VMEM_SHARED((16 * _OUT,), jnp.float32),
        ],
    )(_sc_body)
    return sc_call(inp, rows, cols, vals)


# speedup vs baseline: 1.0037x; 1.0037x over previous
import functools

import numpy as np
import jax
import jax.numpy as jnp
from jax import lax
from jax.experimental import pallas as pl
from jax.experimental.pallas import tpu as pltpu
from jax.experimental.pallas import tpu_sc as plsc

_B = 32
_IN = 4096
_OUT = 4096
_K = 128
_EPS = 1e-6
_P1 = 32749
_P2 = 32719


def _threefry_raw(keypair, x0, x1):
    x0 = x0.copy()
    x1 = x1.copy()
    ks0 = np.uint32(keypair[0])
    ks1 = np.uint32(keypair[1])
    ks2 = np.uint32(ks0 ^ ks1 ^ np.uint32(0x1BD11BDA))
    rot = [np.uint32(r) for r in (13, 15, 26, 6, 17, 29, 16, 24)]

    def rotl(x, d):
        return (x << d) | (x >> np.uint32(32 - int(d)))

    x0 += ks0
    x1 += ks1
    ks = [ks1, ks2, ks2, ks0, ks0, ks1, ks1, ks2, ks2, ks0]
    for g in range(5):
        for r in (rot[0:4] if g % 2 == 0 else rot[4:8]):
            x0 += x1
            x1 = rotl(x1, r)
            x1 ^= x0
        x0 += ks[2 * g]
        x1 += ks[2 * g + 1] + np.uint32(g + 1)
    return x0, x1


def _uniform01(keypair, n):
    b1, b2 = _threefry_raw(keypair, np.zeros(n, np.uint32),
                           np.arange(n, dtype=np.uint32))
    bits = b1 ^ b2
    return ((bits >> np.uint32(9)) | np.uint32(0x3F800000)).view(np.float32) \
        - np.float32(1.0)


def _gen_consts():
    err = np.seterr(over="ignore")
    key1 = np.array([0, 1], np.uint32)
    s1, s2 = _threefry_raw(key1, np.zeros(2, np.uint32),
                           np.arange(2, dtype=np.uint32))
    kr = np.array([s1[0], s2[0]], np.uint32)
    kg = np.array([s1[1], s2[1]], np.uint32)
    n = _B * _K * 4 * 2
    rr = (_uniform01(kr, n) * np.float32(1.0 - _EPS)).reshape(_B, _K, 4, 2)
    gs = (_uniform01(kg, n) * np.float32(1.0 - _EPS)).reshape(_B, _K, 4, 2)
    np.seterr(**err)
    gsf = gs * np.float32(4096.0)
    smp = np.floor(gsf).astype(np.int32)
    return (
        np.ascontiguousarray(rr[..., 0].transpose(0, 2, 1)),
        np.ascontiguousarray(rr[..., 1].transpose(0, 2, 1)),
        np.ascontiguousarray(smp[..., 0].transpose(0, 2, 1)),
        np.ascontiguousarray(smp[..., 1].transpose(0, 2, 1)),
    )


_RR_R, _RR_C, _SMP_R, _SMP_C = _gen_consts()


def _hyper_body(inp_ref, w_ref, b_ref, mr_ref, mc_ref, inv_ref, val_ref):
    res = jnp.dot(inp_ref[...], w_ref[...], preferred_element_type=jnp.float32)
    res = res + b_ref[...]
    x_mr = res[:, 0:128]
    x_mc = res[:, 128:256]
    x_sg = res[:, 256:384]
    x_v = res[:, 384:512]
    mr_ref[...] = 4095.0 / (1.0 + jnp.exp(-x_mr))
    mc_ref[...] = 4095.0 / (1.0 + jnp.exp(-x_mc))
    xs = x_sg + 2.0
    sp = jnp.maximum(xs, 0.0) + jnp.log(1.0 + jnp.exp(-jnp.abs(xs)))
    sig = (sp + _EPS) * 4096.0
    inv_ref[...] = 1.0 / (_EPS + sig)
    val_ref[...] = x_v


def _points_body(mr_ref, mc_ref, inv_ref, val_ref, sr_ref, sc_ref, rr_ref, rc_ref,
                 pk_ref, vals_ref, props_scr):
    b = pl.program_id(0)
    mr = mr_ref[0]
    mc = mc_ref[0]
    inv = inv_ref[0]
    w0 = val_ref[0]

    fr = jnp.floor(mr)
    cr = jnp.ceil(mr)
    fc = jnp.floor(mc)
    cc = jnp.ceil(mc)
    mnsr = jnp.round(mr)
    mnsc = jnp.round(mc)
    lo_r = jnp.maximum(mnsr - 32.0, 0.0)
    lo_r = jnp.where(mnsr + 32.0 > 4096.0, 4096.0 - 64.0, lo_r)
    lo_c = jnp.maximum(mnsc - 32.0, 0.0)
    lo_c = jnp.where(mnsc + 32.0 > 4096.0, 4096.0 - 64.0, lo_c)
    reg_r = rr_ref[0] * 64.0 + lo_r
    reg_c = rc_ref[0] * 64.0 + lo_c

    nbr_r = jnp.concatenate([fr, fr, cr, cr], axis=0).astype(jnp.int32)
    nbr_c = jnp.concatenate([fc, cc, fc, cc], axis=0).astype(jnp.int32)
    R = jnp.concatenate([nbr_r, sr_ref[0], reg_r.astype(jnp.int32)], axis=0)
    C = jnp.concatenate([nbr_c, sc_ref[0], reg_c.astype(jnp.int32)], axis=0)

    rp = R + 1
    cp = C + 1
    k1 = (rp * rp) * ((cp * cp) * cp)
    a1 = (rp * rp) % _P1
    t1 = (((cp * cp) % _P1) * cp) % _P1
    k2 = (a1 * t1) % _P1
    a2 = (rp * rp) % _P2
    t2 = (((cp * cp) % _P2) * cp) % _P2
    k3 = (a2 * t2) % _P2
    k23 = k2 * 32768 + k3

    tri = lax.broadcasted_iota(jnp.int32, (_K, _K), 1) > \
        lax.broadcasted_iota(jnp.int32, (_K, _K), 0)
    k1l = [k1[s][None, :] for s in range(12)]
    k23l = [k23[s][None, :] for s in range(12)]
    k1s = [k1[s][:, None] for s in range(12)]
    k23s = [k23[s][:, None] for s in range(12)]
    dup_rows = []
    for s1 in range(12):
        acc = None
        for s2 in range(s1 + 1):
            eq = (k1l[s1] == k1s[s2]) & (k23l[s1] == k23s[s2])
            if s2 == s1:
                eq = eq & tri
            acc = eq if acc is None else (acc | eq)
        hit = jnp.where(acc, jnp.int32(1), jnp.int32(0))
        dup_rows.append(jnp.max(hit, axis=0))

    Rf = R.astype(jnp.float32)
    Cf = C.astype(jnp.float32)
    S = jnp.zeros((1, 128), jnp.float32)
    for s in range(12):
        dr = Rf[s][:, None] - mr
        dc = Cf[s][:, None] - mc
        p = jnp.exp(-0.5 * (dr * dr + dc * dc) * inv)
        p = jnp.where(dup_rows[s][:, None] > 0, 0.0, p)
        props_scr[s] = p
        S = S + jnp.sum(p, axis=0, keepdims=True)
    w = w0 / (S + _EPS)
    vlist = [jnp.sum(props_scr[s] * w, axis=1)[None, :] for s in range(12)]

    soff = (b // 2) * 4096
    pk_ref[...] = jnp.concatenate([R + soff, C + soff], axis=0)
    vals_ref[...] = jnp.concatenate(
        vlist + [jnp.zeros((4, _K), jnp.float32)], axis=0)


def _sc_body(inp_hbm, pk_hbm, vals_hbm, out_hbm, pk_v, vals_v, gath_v, y_v,
             inp_sh, y_sh):
    c = lax.axis_index("c")
    s = lax.axis_index("s")
    b = s * jnp.int32(2) + c
    soff = s * jnp.int32(4096)
    pltpu.sync_copy(inp_hbm.at[b], inp_sh.at[pl.ds(soff, 4096)])
    pltpu.sync_copy(pk_hbm.at[pl.ds(b * jnp.int32(24), 24)], pk_v)
    pltpu.sync_copy(vals_hbm.at[pl.ds(b * jnp.int32(16), 16)], vals_v)
    for j in range(12):
        pltpu.sync_copy(inp_sh.at[pk_v.at[jnp.int32(12 + j)]],
                        gath_v.at[jnp.int32(j)])
    for j in range(12):
        for t in range(8):
            ix = (jnp.int32(j), pl.ds(t * 16, 16))
            gath_v[ix] = gath_v[ix] * vals_v[ix]

    def zero(i, carry):
        y_v[pl.ds(i * jnp.int32(16), 16)] = jnp.zeros((16,), jnp.float32)
        return carry

    lax.fori_loop(jnp.int32(0), jnp.int32(256), zero, jnp.int32(0))
    pltpu.sync_copy(y_v, y_sh.at[pl.ds(soff, 4096)])
    for j in range(12):
        pltpu.sync_copy(gath_v.at[jnp.int32(j)],
                        y_sh.at[pk_v.at[jnp.int32(j)]], add=True)
    pltpu.sync_copy(y_sh.at[pl.ds(soff, 4096)], out_hbm.at[b])


def kernel(input, W_hyper, b_hyper):
    inp = input.astype(jnp.float32)
    Wp = W_hyper.astype(jnp.float32).reshape(_IN, _K, 4).transpose(0, 2, 1).reshape(_IN, 4 * _K)
    bp = b_hyper.astype(jnp.float32).reshape(_K, 4).T.reshape(1, 4 * _K)

    mr, mc, inv, val = pl.pallas_call(
        _hyper_body,
        out_shape=[jax.ShapeDtypeStruct((_B, _K), jnp.float32)] * 4,
    )(inp, Wp, bp)

    sr = jnp.asarray(_SMP_R)
    sc = jnp.asarray(_SMP_C)
    rrr = jnp.asarray(_RR_R)
    rrc = jnp.asarray(_RR_C)

    _z = np.int32(0)
    vec_spec = pl.BlockSpec((1, 1, _K), lambda b: (b, _z, _z))
    slot_spec = pl.BlockSpec((1, 4, _K), lambda b: (b, _z, _z))
    packed, vals = pl.pallas_call(
        _points_body,
        grid=(_B,),
        in_specs=[vec_spec, vec_spec, vec_spec, vec_spec,
                  slot_spec, slot_spec, slot_spec, slot_spec],
        out_specs=[pl.BlockSpec((24, _K), lambda b: (b, _z)),
                   pl.BlockSpec((16, _K), lambda b: (b, _z))],
        out_shape=[jax.ShapeDtypeStruct((_B * 24, _K), jnp.int32),
                   jax.ShapeDtypeStruct((_B * 16, _K), jnp.float32)],
        scratch_shapes=[pltpu.VMEM((12, _K, _K), jnp.float32)],
    )(mr.reshape(_B, 1, _K), mc.reshape(_B, 1, _K),
      inv.reshape(_B, 1, _K), val.reshape(_B, 1, _K), sr, sc, rrr, rrc)

    sc_call = functools.partial(
        pl.kernel,
        mesh=plsc.VectorSubcoreMesh(core_axis_name="c", subcore_axis_name="s"),
        out_type=jax.ShapeDtypeStruct((_B, _OUT), jnp.float32),
        scratch_types=[
            pltpu.VMEM((24, _K), jnp.int32),
            pltpu.VMEM((16, _K), jnp.float32),
            pltpu.VMEM((12, _K), jnp.float32),
            pltpu.VMEM((_OUT,), jnp.float32),
            pltpu.VMEM_SHARED((16 * _IN,), jnp.float32),
            pltpu.VMEM_SHARED((16 * _OUT,), jnp.float32),
        ],
    )(_sc_body)
    return sc_call(inp, packed, vals)
